# baseline (speedup 1.0000x reference)
import jax
import jax.numpy as jnp
from jax.experimental import pallas as pl
from jax.experimental.pallas import tpu as pltpu

_N = 4096
_R = 512
_G = _N // _R
_RM = 256
_GM = _N // _RM


def _leaky(z):
    return jnp.where(z >= 0, z, 0.2 * z)


def _dot(a, b):
    return jnp.dot(a, b, preferred_element_type=jnp.float32)


def _soft_assign(z, cluster):
    zn = jnp.sum(z * z, axis=1, keepdims=True)
    cn = jnp.sum(cluster * cluster, axis=1)[None, :]
    d2 = zn + cn - 2.0 * _dot(z, cluster.T)
    q = 1.0 / (1.0 + d2)
    return q / jnp.sum(q, axis=1, keepdims=True)


def _pre_kernel(x_ref, w0, b0, w1, b1, w2, b2, w3, b3,
                gw0, gw1, gw2, cl, zae_out, q1_out, v0_out):
    x = x_ref[...]
    z = _leaky(_dot(x, w0[...]) + b0[...])
    z = _leaky(_dot(z, w1[...]) + b1[...])
    z = _leaky(_dot(z, w2[...]) + b2[...])
    zae = _dot(z, w3[...]) + b3[...]
    zae_out[...] = zae
    q1_out[...] = _soft_assign(zae, cl[...])
    wg = _dot(_dot(gw0[...], gw1[...]), gw2[...])
    v0_out[...] = _dot(x, wg)


def _spmm_cast_kernel(adj_ref, v_ref, o_ref, adjbf_ref):
    a = adj_ref[...]
    adjbf_ref[...] = a.astype(jnp.bfloat16)
    o_ref[...] = _dot(a, v_ref[...])


def _mega_kernel(adjbf_ref, v1_ref, zae_ref, a_ref, gamma_ref, cl_ref,
                 dw0, db0, dw1, db1, dw2, db2, dw3, db3,
                 gw0, gw1, gw2,
                 zs_out, q2_out, zt_out, t3_out, tp_out,
                 zhat_out, xhat_out, q_out,
                 va, vb):
    s = pl.program_id(0)
    r = pl.program_id(1)
    rs = pl.ds(r * _RM, _RM)
    adjb = adjbf_ref[...]
    cl = cl_ref[...]

    @pl.when(s == 0)
    def _():
        va[rs, :] = _dot(adjb, v1_ref[...].astype(jnp.bfloat16))

    @pl.when(s == 1)
    def _():
        zs_r = _dot(adjb, va[...].astype(jnp.bfloat16))
        zs_out[rs, :] = zs_r
        q2_out[rs, :] = _soft_assign(zs_r, cl)
        a_r = a_ref[rs, :]
        vb[rs, :] = a_r * zae_ref[rs, :] + (1.0 - a_r) * zs_r

    @pl.when(s == 2)
    def _():
        va[rs, :] = _dot(adjb, vb[...].astype(jnp.bfloat16))

    @pl.when(s == 3)
    def _():
        zl_r = va[rs, :]
        zl = va[...]
        sc = _dot(zl_r, zl.T)
        m = jnp.max(sc, axis=1, keepdims=True)
        p = jnp.exp(sc - m)
        zg = _dot(p, zl) / jnp.sum(p, axis=1, keepdims=True)
        zt_r = gamma_ref[0, 0] * zg + zl_r
        zt_out[rs, :] = zt_r
        vb[rs, :] = zt_r

    @pl.when(s == 4)
    def _():
        va[rs, :] = _dot(adjb, vb[...].astype(jnp.bfloat16))

    @pl.when(s == 5)
    def _():
        vb[rs, :] = _dot(adjb, va[...].astype(jnp.bfloat16))

    @pl.when(s == 6)
    def _():
        t3_r = _dot(adjb, vb[...].astype(jnp.bfloat16))
        t3_out[rs, :] = t3_r
        ug = _dot(_dot(gw0[...], gw1[...]), gw2[...])
        zhat_out[rs, :] = _dot(t3_r, ug)
        tp_out[rs, :] = _dot(t3_r, _dot(ug, ug.T))
        zt_r = zt_out[rs, :]
        d = _leaky(_dot(zt_r, dw0[...]) + db0[...])
        d = _leaky(_dot(d, dw1[...]) + db1[...])
        d = _leaky(_dot(d, dw2[...]) + db2[...])
        xhat_out[rs, :] = _dot(d, dw3[...]) + db3[...]
        q_out[rs, :] = _soft_assign(zt_r, cl)


def _adjhat_kernel(zs_r_ref, zs_ref, tp_ref, t3_ref, o_ref):
    a1 = _dot(zs_r_ref[...], zs_ref[...].T)
    a2 = _dot(tp_ref[...], t3_ref[...].T)
    o_ref[...] = jax.nn.sigmoid(a1) + jax.nn.sigmoid(a2)


def _full(arr):
    nd = arr.ndim
    return pl.BlockSpec(arr.shape, lambda i, _n=nd: (0,) * _n)


def _row(last):
    return pl.BlockSpec((_R, last), lambda i: (i, 0))


def _sds(shape):
    return jax.ShapeDtypeStruct(shape, jnp.float32)


def kernel(x, adj, params):
    p = params
    b = {k: p[k].reshape(1, -1) for k in p if k.startswith('ae_') and '_b' in k}
    gamma = p['gamma'].reshape(1, 1)
    cl = p['cluster']

    zae, q1, v0 = pl.pallas_call(
        _pre_kernel,
        grid=(_G,),
        in_specs=[_row(512),
                  _full(p['ae_enc_w0']), _full(b['ae_enc_b0']),
                  _full(p['ae_enc_w1']), _full(b['ae_enc_b1']),
                  _full(p['ae_enc_w2']), _full(b['ae_enc_b2']),
                  _full(p['ae_enc_w3']), _full(b['ae_enc_b3']),
                  _full(p['gae_enc_w0']), _full(p['gae_enc_w1']),
                  _full(p['gae_enc_w2']), _full(cl)],
        out_specs=[_row(32), _row(10), _row(32)],
        out_shape=[_sds((_N, 32)), _sds((_N, 10)), _sds((_N, 32))],
    )(x, p['ae_enc_w0'], b['ae_enc_b0'], p['ae_enc_w1'], b['ae_enc_b1'],
      p['ae_enc_w2'], b['ae_enc_b2'], p['ae_enc_w3'], b['ae_enc_b3'],
      p['gae_enc_w0'], p['gae_enc_w1'], p['gae_enc_w2'], cl)

    v1, adj_bf = pl.pallas_call(
        _spmm_cast_kernel,
        grid=(_G,),
        in_specs=[_row(_N), _full(v0)],
        out_specs=[_row(32), _row(_N)],
        out_shape=[_sds((_N, 32)),
                   jax.ShapeDtypeStruct((_N, _N), jnp.bfloat16)],
    )(adj, v0)

    def cfull(shape):
        return pl.BlockSpec(shape, lambda s_, r_: (0,) * len(shape))

    adjbf_spec = pl.BlockSpec(
        (_RM, _N), lambda s_, r_: (jnp.where(s_ == 3, 0, r_), 0))

    mega_ins = [v1, zae, p['a'], gamma, cl,
                p['ae_dec_w0'], b['ae_dec_b0'], p['ae_dec_w1'], b['ae_dec_b1'],
                p['ae_dec_w2'], b['ae_dec_b2'], p['ae_dec_w3'], b['ae_dec_b3'],
                p['gae_dec_w0'], p['gae_dec_w1'], p['gae_dec_w2']]
    zs, q2, zt, t3, tp, zhat, xhat, q = pl.pallas_call(
        _mega_kernel,
        grid=(7, _GM),
        in_specs=[adjbf_spec] + [cfull(t.shape) for t in mega_ins],
        out_specs=[cfull((_N, 32)), cfull((_N, 10)), cfull((_N, 32)),
                   cfull((_N, 32)), cfull((_N, 32)), cfull((_N, 512)),
                   cfull((_N, 512)), cfull((_N, 10))],
        out_shape=[_sds((_N, 32)), _sds((_N, 10)), _sds((_N, 32)),
                   _sds((_N, 32)), _sds((_N, 32)), _sds((_N, 512)),
                   _sds((_N, 512)), _sds((_N, 10))],
        scratch_shapes=[pltpu.VMEM((_N, 32), jnp.float32),
                        pltpu.VMEM((_N, 32), jnp.float32)],
    )(adj_bf, *mega_ins)

    adj_hat = pl.pallas_call(
        _adjhat_kernel,
        grid=(_G,),
        in_specs=[_row(32), _full(zs), _row(32), _full(t3)],
        out_specs=_row(_N),
        out_shape=_sds((_N, _N)),
    )(zs, zs, tp, t3)

    return (xhat, zhat, adj_hat, zae, zs, q, q1, q2, zt)


# speedup vs baseline: 1.1342x; 1.1342x over previous
import jax
import jax.numpy as jnp
from jax.experimental import pallas as pl
from jax.experimental.pallas import tpu as pltpu

_N = 4096
_R = 512
_G = _N // _R
_RM = 256
_GM = _N // _RM


def _leaky(z):
    return jnp.where(z >= 0, z, 0.2 * z)


def _dot(a, b):
    return jnp.dot(a, b, preferred_element_type=jnp.float32)


def _soft_assign(z, cluster):
    zn = jnp.sum(z * z, axis=1, keepdims=True)
    cn = jnp.sum(cluster * cluster, axis=1)[None, :]
    d2 = zn + cn - 2.0 * _dot(z, cluster.T)
    q = 1.0 / (1.0 + d2)
    return q / jnp.sum(q, axis=1, keepdims=True)


def _pre_kernel(x_ref, w0, b0, w1, b1, w2, b2, w3, b3,
                gw0, gw1, gw2, cl, zae_out, q1_out, v0_out):
    x = x_ref[...]
    z = _leaky(_dot(x, w0[...]) + b0[...])
    z = _leaky(_dot(z, w1[...]) + b1[...])
    z = _leaky(_dot(z, w2[...]) + b2[...])
    zae = _dot(z, w3[...]) + b3[...]
    zae_out[...] = zae
    q1_out[...] = _soft_assign(zae, cl[...])
    wg = _dot(_dot(gw0[...], gw1[...]), gw2[...])
    v0_out[...] = _dot(x, wg)


def _spmm_cast_kernel(adj_ref, v_ref, o_ref, adjbf_ref):
    a = adj_ref[...]
    adjbf_ref[...] = a.astype(jnp.bfloat16)
    o_ref[...] = _dot(a, v_ref[...])


def _mega_kernel(adjbf_ref, v1_ref, zae_ref, a_ref, gamma_ref,
                 zs_out, zt_out, t3_out, va, vb):
    s = pl.program_id(0)
    r = pl.program_id(1)
    rs = pl.ds(r * _R, _R)
    adjb = adjbf_ref[...]

    @pl.when(s == 0)
    def _():
        va[rs, :] = _dot(adjb, v1_ref[...].astype(jnp.bfloat16))

    @pl.when(s == 1)
    def _():
        zs_r = _dot(adjb, va[...].astype(jnp.bfloat16))
        zs_out[rs, :] = zs_r
        a_r = a_ref[rs, :]
        vb[rs, :] = a_r * zae_ref[rs, :] + (1.0 - a_r) * zs_r

    @pl.when(s == 2)
    def _():
        va[rs, :] = _dot(adjb, vb[...].astype(jnp.bfloat16))

    @pl.when(s == 3)
    def _():
        zl_r = va[rs, :]
        zl = va[...]
        sc = _dot(zl_r, zl.T)
        m = jnp.max(sc, axis=1, keepdims=True)
        p = jnp.exp(sc - m)
        zg = _dot(p, zl) / jnp.sum(p, axis=1, keepdims=True)
        zt_r = gamma_ref[0, 0] * zg + zl_r
        zt_out[rs, :] = zt_r
        vb[rs, :] = zt_r

    @pl.when(s == 4)
    def _():
        va[rs, :] = _dot(adjb, vb[...].astype(jnp.bfloat16))

    @pl.when(s == 5)
    def _():
        vb[rs, :] = _dot(adjb, va[...].astype(jnp.bfloat16))

    @pl.when(s == 6)
    def _():
        t3_out[rs, :] = _dot(adjb, vb[...].astype(jnp.bfloat16))


def _tail_kernel(zt_ref, t3_ref, zs_ref,
                 dw0, db0, dw1, db1, dw2, db2, dw3, db3,
                 gw0, gw1, gw2, cl,
                 xhat_out, zhat_out, q_out, q2_out, tp_out):
    zt = zt_ref[...]
    d = _leaky(_dot(zt, dw0[...]) + db0[...])
    d = _leaky(_dot(d, dw1[...]) + db1[...])
    d = _leaky(_dot(d, dw2[...]) + db2[...])
    xhat_out[...] = _dot(d, dw3[...]) + db3[...]
    ug = _dot(_dot(gw0[...], gw1[...]), gw2[...])
    t3 = t3_ref[...]
    zhat_out[...] = _dot(t3, ug)
    tp_out[...] = _dot(t3, _dot(ug, ug.T))
    q_out[...] = _soft_assign(zt, cl[...])
    q2_out[...] = _soft_assign(zs_ref[...], cl[...])


def _adjhat_kernel(zs_r_ref, zs_ref, tp_ref, t3_ref, o_ref):
    a1 = _dot(zs_r_ref[...], zs_ref[...].T)
    a2 = _dot(tp_ref[...], t3_ref[...].T)
    o_ref[...] = jax.nn.sigmoid(a1) + jax.nn.sigmoid(a2)


def _full(arr):
    nd = arr.ndim
    return pl.BlockSpec(arr.shape, lambda i, _n=nd: (0,) * _n)


def _row(last):
    return pl.BlockSpec((_R, last), lambda i: (i, 0))


def _sds(shape):
    return jax.ShapeDtypeStruct(shape, jnp.float32)


def kernel(x, adj, params):
    p = params
    b = {k: p[k].reshape(1, -1) for k in p if k.startswith('ae_') and '_b' in k}
    gamma = p['gamma'].reshape(1, 1)
    cl = p['cluster']

    zae, q1, v0 = pl.pallas_call(
        _pre_kernel,
        grid=(_G,),
        in_specs=[_row(512),
                  _full(p['ae_enc_w0']), _full(b['ae_enc_b0']),
                  _full(p['ae_enc_w1']), _full(b['ae_enc_b1']),
                  _full(p['ae_enc_w2']), _full(b['ae_enc_b2']),
                  _full(p['ae_enc_w3']), _full(b['ae_enc_b3']),
                  _full(p['gae_enc_w0']), _full(p['gae_enc_w1']),
                  _full(p['gae_enc_w2']), _full(cl)],
        out_specs=[_row(32), _row(10), _row(32)],
        out_shape=[_sds((_N, 32)), _sds((_N, 10)), _sds((_N, 32))],
    )(x, p['ae_enc_w0'], b['ae_enc_b0'], p['ae_enc_w1'], b['ae_enc_b1'],
      p['ae_enc_w2'], b['ae_enc_b2'], p['ae_enc_w3'], b['ae_enc_b3'],
      p['gae_enc_w0'], p['gae_enc_w1'], p['gae_enc_w2'], cl)

    v1, adj_bf = pl.pallas_call(
        _spmm_cast_kernel,
        grid=(_G,),
        in_specs=[_row(_N), _full(v0)],
        out_specs=[_row(32), _row(_N)],
        out_shape=[_sds((_N, 32)),
                   jax.ShapeDtypeStruct((_N, _N), jnp.bfloat16)],
    )(adj, v0)

    def cfull(shape):
        return pl.BlockSpec(shape, lambda s_, r_: (0,) * len(shape))

    adjbf_spec = pl.BlockSpec(
        (_R, _N), lambda s_, r_: (jnp.where(s_ == 3, 0, r_), 0))

    mega_ins = [v1, zae, p['a'], gamma]
    zs, zt, t3 = pl.pallas_call(
        _mega_kernel,
        grid=(7, _G),
        in_specs=[adjbf_spec] + [cfull(t.shape) for t in mega_ins],
        out_specs=[cfull((_N, 32)), cfull((_N, 32)), cfull((_N, 32))],
        out_shape=[_sds((_N, 32)), _sds((_N, 32)), _sds((_N, 32))],
        scratch_shapes=[pltpu.VMEM((_N, 32), jnp.float32),
                        pltpu.VMEM((_N, 32), jnp.float32)],
    )(adj_bf, *mega_ins)

    xhat, zhat, q, q2, tp = pl.pallas_call(
        _tail_kernel,
        grid=(_G,),
        in_specs=[_row(32), _row(32), _row(32),
                  _full(p['ae_dec_w0']), _full(b['ae_dec_b0']),
                  _full(p['ae_dec_w1']), _full(b['ae_dec_b1']),
                  _full(p['ae_dec_w2']), _full(b['ae_dec_b2']),
                  _full(p['ae_dec_w3']), _full(b['ae_dec_b3']),
                  _full(p['gae_dec_w0']), _full(p['gae_dec_w1']),
                  _full(p['gae_dec_w2']), _full(cl)],
        out_specs=[_row(512), _row(512), _row(10), _row(10), _row(32)],
        out_shape=[_sds((_N, 512)), _sds((_N, 512)), _sds((_N, 10)),
                   _sds((_N, 10)), _sds((_N, 32))],
    )(zt, t3, zs,
      p['ae_dec_w0'], b['ae_dec_b0'], p['ae_dec_w1'], b['ae_dec_b1'],
      p['ae_dec_w2'], b['ae_dec_b2'], p['ae_dec_w3'], b['ae_dec_b3'],
      p['gae_dec_w0'], p['gae_dec_w1'], p['gae_dec_w2'], cl)

    adj_hat = pl.pallas_call(
        _adjhat_kernel,
        grid=(_G,),
        in_specs=[_row(32), _full(zs), _row(32), _full(t3)],
        out_specs=_row(_N),
        out_shape=_sds((_N, _N)),
    )(zs, zs, tp, t3)

    return (xhat, zhat, adj_hat, zae, zs, q, q1, q2, zt)
